# baseline (speedup 1.0000x reference)
import functools

import jax
import jax.numpy as jnp
from jax import lax
from jax.experimental import pallas as pl
from jax.experimental.pallas import tpu as pltpu
from jax.experimental.pallas import tpu_sc as plsc

B, T, C, E, TOPK = 2, 2048, 1024, 8, 2
H = 4 * C
N = B * T
A = N * TOPK
M = 256
NB = A // M + E - 1
NR = NB * M

SC_CORES, SC_SUBCORES = 2, 16
NW = SC_CORES * SC_SUBCORES
G_ROWS = NR // NW
G_CH = 40
G_CHUNKS = [40, 40, 40, 40, 40, 40, 40, 32]
C_TOK = N // NW
C_CH = 24
C_CHUNKS = [24, 24, 24, 24, 24, 8]
CW = C // 2

RBLK = 512


def _router_body(x_ref, wg_ref, lg_ref, i1_ref, i2_ref, w1_ref, w2_ref):
    xb = x_ref[...]
    lg = lax.dot_general(xb, wg_ref[...], (((1,), (1,)), ((), ())),
                         preferred_element_type=jnp.float32)
    lg_ref[...] = lg
    j = lax.broadcasted_iota(jnp.int32, lg.shape, 1)
    m1 = jnp.max(lg, axis=1, keepdims=True)
    i1 = jnp.min(jnp.where(lg == m1, j, E), axis=1)
    lg2 = jnp.where(j == i1[:, None], -jnp.inf, lg)
    m2 = jnp.max(lg2, axis=1, keepdims=True)
    i2 = jnp.min(jnp.where(lg2 == m2, j, E), axis=1)
    w1 = 1.0 / (1.0 + jnp.exp(m2 - m1))
    i1_ref[...] = i1[:, None]
    i2_ref[...] = i2[:, None]
    w1_ref[...] = w1
    w2_ref[...] = 1.0 - w1


def _mlp_body(meta_ref, xs_ref, wfc_ref, wpj_ref, w_ref, ys_ref):
    b = pl.program_id(0)
    valid = meta_ref[NB + b] == 1

    @pl.when(valid)
    def _():
        xb = xs_ref[...]
        h = lax.dot_general(xb, wfc_ref[0], (((1,), (1,)), ((), ())),
                            preferred_element_type=jnp.float32)
        a = jnp.square(jnp.maximum(h, 0.0)).astype(jnp.bfloat16)
        y = lax.dot_general(a, wpj_ref[0], (((1,), (1,)), ((), ())),
                            preferred_element_type=jnp.float32)
        ys_ref[...] = y * w_ref[...]


def _sc_gather_body(tok_hbm, x_hbm, out_hbm, idx0, idx1, buf0, buf1,
                    gsem0, gsem1, osem0, osem1):
    wid = lax.axis_index("s") * SC_CORES + lax.axis_index("c")
    base = pl.multiple_of(wid * G_ROWS, 8)
    idxs, bufs = (idx0, idx1), (buf0, buf1)
    gsems, osems = (gsem0, gsem1), (osem0, osem1)
    offs = [0]
    for n in G_CHUNKS:
        offs.append(offs[-1] + n)
    gathers = [None, None]
    writes = [None, None]
    for cidx, nrow in enumerate(G_CHUNKS):
        s = cidx % 2
        off = base + offs[cidx]
        if writes[s] is not None:
            writes[s].wait()
        pltpu.sync_copy(tok_hbm.at[pl.ds(off, nrow)], idxs[s].at[pl.ds(0, nrow)])
        gathers[s] = pltpu.async_copy(
            x_hbm.at[idxs[s].at[pl.ds(0, nrow)]], bufs[s].at[pl.ds(0, nrow)],
            gsems[s])
        gathers[s].wait()
        writes[s] = pltpu.async_copy(
            bufs[s].at[pl.ds(0, nrow)], out_hbm.at[pl.ds(off, nrow)], osems[s])
    for w in writes:
        if w is not None:
            w.wait()


def _sc_combine_body(p1_hbm, p2_hbm, ys_hbm, out_hbm, ia0, ib0, ia1, ib1,
                     ba0, bb0, ba1, bb1, sa0, sb0, sa1, sb1, os0, os1):
    wid = lax.axis_index("s") * SC_CORES + lax.axis_index("c")
    base = pl.multiple_of(wid * C_TOK, 8)
    ias, ibs = (ia0, ia1), (ib0, ib1)
    bas, bbs = (ba0, ba1), (bb0, bb1)
    sas, sbs, oss = (sa0, sa1), (sb0, sb1), (os0, os1)
    writes = [None, None]
    offs = [0]
    for n in C_CHUNKS:
        offs.append(offs[-1] + n)
    for cidx, nrow in enumerate(C_CHUNKS):
        s = cidx % 2
        off = base + offs[cidx]
        if writes[s] is not None:
            writes[s].wait()
        pltpu.sync_copy(p1_hbm.at[pl.ds(off, nrow)], ias[s].at[pl.ds(0, nrow)])
        pltpu.sync_copy(p2_hbm.at[pl.ds(off, nrow)], ibs[s].at[pl.ds(0, nrow)])
        cpa = pltpu.async_copy(ys_hbm.at[ias[s].at[pl.ds(0, nrow)]],
                               bas[s].at[pl.ds(0, nrow)], sas[s])
        cpb = pltpu.async_copy(ys_hbm.at[ibs[s].at[pl.ds(0, nrow)]],
                               bbs[s].at[pl.ds(0, nrow)], sbs[s])
        cpa.wait()
        cpb.wait()
        for r in range(nrow):
            def add_col(jc, _, r=r, s=s):
                sl = pl.ds(jc * 16, 16)
                bas[s][r, sl] = bas[s][r, sl] + bbs[s][r, sl]
                return 0
            lax.fori_loop(0, C // 16, add_col, 0)
        writes[s] = pltpu.async_copy(bas[s].at[pl.ds(0, nrow)],
                                     out_hbm.at[pl.ds(off, nrow)], oss[s])
    for w in writes:
        if w is not None:
            w.wait()


def _routing_metadata(i1, i2, w1, w2):
    ea = jnp.concatenate([i1, i2])
    wa = jnp.concatenate([w1, w2])
    ta = jnp.tile(jnp.arange(N, dtype=jnp.int32), 2)
    onehot = ea[:, None] == jnp.arange(E, dtype=jnp.int32)[None, :]
    cnt = jnp.sum(onehot, axis=0, dtype=jnp.int32)
    rank = jnp.cumsum(onehot.astype(jnp.int32), axis=0) - 1
    r_a = jnp.sum(jnp.where(onehot, rank, 0), axis=1)
    nblk = (cnt + M - 1) // M
    blk_start = jnp.cumsum(nblk) - nblk
    pos = blk_start[ea] * M + r_a
    row_token = jnp.zeros((NR,), jnp.int32).at[pos].set(ta)
    row_weight = jnp.zeros((NR,), jnp.float32).at[pos].set(wa)
    total_blk = blk_start[-1] + nblk[-1]
    bidx = jnp.arange(NB, dtype=jnp.int32)
    bc = jnp.minimum(bidx, total_blk - 1)
    bg = jnp.sum(blk_start[None, :] <= bc[:, None], axis=1,
                 dtype=jnp.int32) - 1
    bvalid = (bidx < total_blk).astype(jnp.int32)
    meta = jnp.concatenate([bg, bvalid])
    pos1, pos2 = pos[:N], pos[N:]
    return meta, row_token, row_weight, pos1, pos2


def kernel(x, W_gate, W_fc, W_proj):
    xf = x.reshape(N, C)
    wfc_bf = W_fc.astype(jnp.bfloat16)
    wpj_bf = W_proj.astype(jnp.bfloat16)

    logits, i1, i2, w1, w2 = pl.pallas_call(
        _router_body,
        grid=(N // RBLK,),
        in_specs=[
            pl.BlockSpec((RBLK, C), lambda i: (i, 0)),
            pl.BlockSpec((E, C), lambda i: (0, 0)),
        ],
        out_specs=[
            pl.BlockSpec((RBLK, E), lambda i: (i, 0)),
            pl.BlockSpec((RBLK, 1), lambda i: (i, 0)),
            pl.BlockSpec((RBLK, 1), lambda i: (i, 0)),
            pl.BlockSpec((RBLK, 1), lambda i: (i, 0)),
            pl.BlockSpec((RBLK, 1), lambda i: (i, 0)),
        ],
        out_shape=[
            jax.ShapeDtypeStruct((N, E), jnp.float32),
            jax.ShapeDtypeStruct((N, 1), jnp.int32),
            jax.ShapeDtypeStruct((N, 1), jnp.int32),
            jax.ShapeDtypeStruct((N, 1), jnp.float32),
            jax.ShapeDtypeStruct((N, 1), jnp.float32),
        ],
    )(xf, W_gate)

    meta, row_token, row_weight, pos1, pos2 = _routing_metadata(
        i1[:, 0], i2[:, 0], w1[:, 0], w2[:, 0])

    x_bf = xf.astype(jnp.bfloat16)
    x_i = lax.bitcast_convert_type(x_bf.reshape(N, CW, 2), jnp.int32)
    mesh = plsc.VectorSubcoreMesh(core_axis_name="c", subcore_axis_name="s")
    xs_i = pl.kernel(
        _sc_gather_body,
        out_type=jax.ShapeDtypeStruct((NR, CW), jnp.int32),
        mesh=mesh,
        scratch_types=[
            pltpu.VMEM((G_CH,), jnp.int32),
            pltpu.VMEM((G_CH,), jnp.int32),
            pltpu.VMEM((G_CH, CW), jnp.int32),
            pltpu.VMEM((G_CH, CW), jnp.int32),
            pltpu.SemaphoreType.DMA,
            pltpu.SemaphoreType.DMA,
            pltpu.SemaphoreType.DMA,
            pltpu.SemaphoreType.DMA,
        ],
    )(row_token, x_i)
    xs = lax.bitcast_convert_type(xs_i, jnp.bfloat16).reshape(NR, C)

    ys = pl.pallas_call(
        _mlp_body,
        grid_spec=pltpu.PrefetchScalarGridSpec(
            num_scalar_prefetch=1,
            grid=(NB,),
            in_specs=[
                pl.BlockSpec((M, C), lambda b, m: (b, 0)),
                pl.BlockSpec((1, H, C), lambda b, m: (m[b], 0, 0)),
                pl.BlockSpec((1, C, H), lambda b, m: (m[b], 0, 0)),
                pl.BlockSpec((M, 1), lambda b, m: (b, 0)),
            ],
            out_specs=pl.BlockSpec((M, C), lambda b, m: (b, 0)),
        ),
        out_shape=jax.ShapeDtypeStruct((NR, C), jnp.float32),
    )(meta, xs, wfc_bf, wpj_bf, row_weight[:, None])

    out = pl.kernel(
        _sc_combine_body,
        out_type=jax.ShapeDtypeStruct((N, C), jnp.float32),
        mesh=mesh,
        scratch_types=(
            [pltpu.VMEM((C_CH,), jnp.int32)] * 4
            + [pltpu.VMEM((C_CH, C), jnp.float32)] * 4
            + [pltpu.SemaphoreType.DMA] * 6
        ),
    )(pos1, pos2, ys)

    return out.reshape(B, T, C), logits


# speedup vs baseline: 1.4886x; 1.4886x over previous
import functools

import jax
import jax.numpy as jnp
from jax import lax
from jax.experimental import pallas as pl
from jax.experimental.pallas import tpu as pltpu
from jax.experimental.pallas import tpu_sc as plsc

B, T, C, E, TOPK = 2, 2048, 1024, 8, 2
H = 4 * C
N = B * T
A = N * TOPK
M = 256
NB = A // M + E - 1
NR = NB * M

SC_CORES, SC_SUBCORES = 2, 16
NW = SC_CORES * SC_SUBCORES
G_ROWS = NR // NW
G_CH = 24
G_CHUNKS = [24] * 13
G_NBUF = 4
C_TOK = N // NW
C_CH = 16
C_CHUNKS = [16] * 8
C_NBUF = 3

RBLK = 512


def _router_body(x_ref, wg_ref, lg_ref, i1_ref, i2_ref, w1_ref, w2_ref):
    xb = x_ref[...]
    lg = lax.dot_general(xb, wg_ref[...], (((1,), (1,)), ((), ())),
                         preferred_element_type=jnp.float32)
    lg_ref[...] = lg
    j = lax.broadcasted_iota(jnp.int32, lg.shape, 1)
    m1 = jnp.max(lg, axis=1, keepdims=True)
    i1 = jnp.min(jnp.where(lg == m1, j, E), axis=1)
    lg2 = jnp.where(j == i1[:, None], -jnp.inf, lg)
    m2 = jnp.max(lg2, axis=1, keepdims=True)
    i2 = jnp.min(jnp.where(lg2 == m2, j, E), axis=1)
    w1 = 1.0 / (1.0 + jnp.exp(m2 - m1))
    i1_ref[...] = i1[:, None]
    i2_ref[...] = i2[:, None]
    w1_ref[...] = w1
    w2_ref[...] = 1.0 - w1


def _mlp_body(meta_ref, xs_ref, wfc_ref, wpj_ref, w_ref, ys_ref):
    b = pl.program_id(0)
    valid = meta_ref[NB + b] == 1

    @pl.when(valid)
    def _():
        xb = xs_ref[...].astype(jnp.bfloat16)
        h = lax.dot_general(xb, wfc_ref[0], (((1,), (1,)), ((), ())),
                            preferred_element_type=jnp.float32)
        a = jnp.square(jnp.maximum(h, 0.0)).astype(jnp.bfloat16)
        y = lax.dot_general(a, wpj_ref[0], (((1,), (1,)), ((), ())),
                            preferred_element_type=jnp.float32)
        ys_ref[...] = y * w_ref[...]


def _sc_gather_body(tok_hbm, x_hbm, out_hbm, *refs):
    idxs = refs[0:G_NBUF]
    bufs = refs[G_NBUF:2 * G_NBUF]
    gsems = refs[2 * G_NBUF:3 * G_NBUF]
    osems = refs[3 * G_NBUF:4 * G_NBUF]
    wid = lax.axis_index("s") * SC_CORES + lax.axis_index("c")
    base = pl.multiple_of(wid * G_ROWS, 8)
    nch = len(G_CHUNKS)
    gathers = [None] * G_NBUF
    writes = [None] * G_NBUF
    lag = G_NBUF - 1

    def issue(cidx):
        s = cidx % G_NBUF
        off = base + cidx * G_CH
        if writes[s] is not None:
            writes[s].wait()
            writes[s] = None
        pltpu.sync_copy(tok_hbm.at[pl.ds(off, G_CH)], idxs[s])
        gathers[s] = pltpu.async_copy(x_hbm.at[idxs[s]], bufs[s], gsems[s])

    def retire(cidx):
        s = cidx % G_NBUF
        off = base + cidx * G_CH
        gathers[s].wait()
        writes[s] = pltpu.async_copy(bufs[s], out_hbm.at[pl.ds(off, G_CH)],
                                     osems[s])

    for cidx in range(nch):
        issue(cidx)
        if cidx >= lag:
            retire(cidx - lag)
    for cidx in range(nch - lag, nch):
        retire(cidx)
    for w in writes:
        if w is not None:
            w.wait()


def _sc_combine_body(p1_hbm, p2_hbm, ys_hbm, out_hbm, *refs):
    ias = refs[0:C_NBUF]
    ibs = refs[C_NBUF:2 * C_NBUF]
    bas = refs[2 * C_NBUF:3 * C_NBUF]
    bbs = refs[3 * C_NBUF:4 * C_NBUF]
    sas = refs[4 * C_NBUF:5 * C_NBUF]
    sbs = refs[5 * C_NBUF:6 * C_NBUF]
    oss = refs[6 * C_NBUF:7 * C_NBUF]
    wid = lax.axis_index("s") * SC_CORES + lax.axis_index("c")
    base = pl.multiple_of(wid * C_TOK, 8)
    nch = len(C_CHUNKS)
    ga = [None] * C_NBUF
    gb = [None] * C_NBUF
    writes = [None] * C_NBUF
    lag = C_NBUF - 1

    def issue(cidx):
        s = cidx % C_NBUF
        off = base + cidx * C_CH
        if writes[s] is not None:
            writes[s].wait()
            writes[s] = None
        pltpu.sync_copy(p1_hbm.at[pl.ds(off, C_CH)], ias[s])
        pltpu.sync_copy(p2_hbm.at[pl.ds(off, C_CH)], ibs[s])
        ga[s] = pltpu.async_copy(ys_hbm.at[ias[s]], bas[s], sas[s])
        gb[s] = pltpu.async_copy(ys_hbm.at[ibs[s]], bbs[s], sbs[s])

    def retire(cidx):
        s = cidx % C_NBUF
        off = base + cidx * C_CH
        ga[s].wait()
        gb[s].wait()
        for r in range(C_CH):
            def add_col(jc, _, r=r, s=s):
                sl = pl.ds(jc * 16, 16)
                bas[s][r, sl] = bas[s][r, sl] + bbs[s][r, sl]
                return 0
            lax.fori_loop(0, C // 16, add_col, 0)
        writes[s] = pltpu.async_copy(bas[s], out_hbm.at[pl.ds(off, C_CH)],
                                     oss[s])

    for cidx in range(nch):
        issue(cidx)
        if cidx >= lag:
            retire(cidx - lag)
    for cidx in range(nch - lag, nch):
        retire(cidx)
    for w in writes:
        if w is not None:
            w.wait()


def _routing_metadata(i1, i2, w1, w2):
    ea = jnp.concatenate([i1, i2])
    wa = jnp.concatenate([w1, w2])
    ta = jnp.tile(jnp.arange(N, dtype=jnp.int32), 2)
    onehot = ea[:, None] == jnp.arange(E, dtype=jnp.int32)[None, :]
    cnt = jnp.sum(onehot, axis=0, dtype=jnp.int32)
    rank = jnp.cumsum(onehot.astype(jnp.int32), axis=0) - 1
    r_a = jnp.sum(jnp.where(onehot, rank, 0), axis=1)
    nblk = (cnt + M - 1) // M
    blk_start = jnp.cumsum(nblk) - nblk
    pos = blk_start[ea] * M + r_a
    row_token = jnp.zeros((NR,), jnp.int32).at[pos].set(ta)
    row_weight = jnp.zeros((NR,), jnp.float32).at[pos].set(wa)
    total_blk = blk_start[-1] + nblk[-1]
    bidx = jnp.arange(NB, dtype=jnp.int32)
    bc = jnp.minimum(bidx, total_blk - 1)
    bg = jnp.sum(blk_start[None, :] <= bc[:, None], axis=1,
                 dtype=jnp.int32) - 1
    bvalid = (bidx < total_blk).astype(jnp.int32)
    meta = jnp.concatenate([bg, bvalid])
    pos1, pos2 = pos[:N], pos[N:]
    return meta, row_token, row_weight, pos1, pos2


def kernel(x, W_gate, W_fc, W_proj):
    xf = x.reshape(N, C)
    wfc_bf = W_fc.astype(jnp.bfloat16)
    wpj_bf = W_proj.astype(jnp.bfloat16)

    logits, i1, i2, w1, w2 = pl.pallas_call(
        _router_body,
        grid=(N // RBLK,),
        in_specs=[
            pl.BlockSpec((RBLK, C), lambda i: (i, 0)),
            pl.BlockSpec((E, C), lambda i: (0, 0)),
        ],
        out_specs=[
            pl.BlockSpec((RBLK, E), lambda i: (i, 0)),
            pl.BlockSpec((RBLK, 1), lambda i: (i, 0)),
            pl.BlockSpec((RBLK, 1), lambda i: (i, 0)),
            pl.BlockSpec((RBLK, 1), lambda i: (i, 0)),
            pl.BlockSpec((RBLK, 1), lambda i: (i, 0)),
        ],
        out_shape=[
            jax.ShapeDtypeStruct((N, E), jnp.float32),
            jax.ShapeDtypeStruct((N, 1), jnp.int32),
            jax.ShapeDtypeStruct((N, 1), jnp.int32),
            jax.ShapeDtypeStruct((N, 1), jnp.float32),
            jax.ShapeDtypeStruct((N, 1), jnp.float32),
        ],
    )(xf, W_gate)

    meta, row_token, row_weight, pos1, pos2 = _routing_metadata(
        i1[:, 0], i2[:, 0], w1[:, 0], w2[:, 0])

    mesh = plsc.VectorSubcoreMesh(core_axis_name="c", subcore_axis_name="s")
    xs = pl.kernel(
        _sc_gather_body,
        out_type=jax.ShapeDtypeStruct((NR, C), jnp.float32),
        mesh=mesh,
        scratch_types=(
            [pltpu.VMEM((G_CH,), jnp.int32)] * G_NBUF
            + [pltpu.VMEM((G_CH, C), jnp.float32)] * G_NBUF
            + [pltpu.SemaphoreType.DMA] * (2 * G_NBUF)
        ),
    )(row_token, xf)

    ys = pl.pallas_call(
        _mlp_body,
        grid_spec=pltpu.PrefetchScalarGridSpec(
            num_scalar_prefetch=1,
            grid=(NB,),
            in_specs=[
                pl.BlockSpec((M, C), lambda b, m: (b, 0)),
                pl.BlockSpec((1, H, C), lambda b, m: (m[b], 0, 0)),
                pl.BlockSpec((1, C, H), lambda b, m: (m[b], 0, 0)),
                pl.BlockSpec((M, 1), lambda b, m: (b, 0)),
            ],
            out_specs=pl.BlockSpec((M, C), lambda b, m: (b, 0)),
        ),
        out_shape=jax.ShapeDtypeStruct((NR, C), jnp.float32),
    )(meta, xs, wfc_bf, wpj_bf, row_weight[:, None])

    out = pl.kernel(
        _sc_combine_body,
        out_type=jax.ShapeDtypeStruct((N, C), jnp.float32),
        mesh=mesh,
        scratch_types=(
            [pltpu.VMEM((C_CH,), jnp.int32)] * (2 * C_NBUF)
            + [pltpu.VMEM((C_CH, C), jnp.float32)] * (2 * C_NBUF)
            + [pltpu.SemaphoreType.DMA] * (3 * C_NBUF)
        ),
    )(pos1, pos2, ys)

    return out.reshape(B, T, C), logits
